# Initial kernel scaffold; baseline (speedup 1.0000x reference)
import jax
import jax.numpy as jnp
from jax.experimental import pallas as pl


def kernel(source_x, source_x_importance, source_edge_index, target_x, target_x_importance, target_edge_index, bn_gamma, bn_beta, W1, b1, W2, b2, W3, b3, W4, b4, W5, b5, Wfc):
    raise NotImplementedError("write your pallas kernel here")


# speedup vs baseline: 6.5607x; 6.5607x over previous
import functools

import jax
import jax.numpy as jnp
from jax import lax
from jax.experimental import pallas as pl
from jax.experimental.pallas import tpu as pltpu
from jax.experimental.pallas import tpu_sc as plsc

_N = 10000
_D = 128
_E = 320000
_NSC = 2
_NSUB = 16
_NW = _NSC * _NSUB
_CS = 128
_CH = 80
_NPH = 2
_HCH = _CH // _NPH
_PW = _CH * _CS
_EPAD = _NW * _PW
_AGG_ROWS = 10240
_ZROWS = _AGG_ROWS // _NSUB


def _sc_agg_body(h_hbm, src_hbm, dst_hbm, zeros_hbm, out_hbm,
                 src_v, dst_v, gb0, gb1, agg_sh, gsem0, gsem1):
    c = lax.axis_index("c")
    s = lax.axis_index("s")
    wid = c * _NSUB + s
    pltpu.sync_copy(zeros_hbm, agg_sh.at[pl.ds(s * _ZROWS, _ZROWS)])
    plsc.subcore_barrier()

    gbufs = (gb0, gb1)
    gsems = (gsem0, gsem1)
    for phase in range(_NPH):
        pltpu.sync_copy(src_hbm.at[wid, pl.ds(phase * _HCH, _HCH)], src_v)
        pltpu.sync_copy(dst_hbm.at[wid, pl.ds(phase * _HCH, _HCH)], dst_v)
        for b in range(2):
            pltpu.async_copy(h_hbm.at[src_v.at[b]], gbufs[b], gsems[b])

        def body(jj, carry):
            for b in range(2):
                j = 2 * jj + b
                pltpu.make_async_copy(h_hbm.at[src_v.at[j]], gbufs[b],
                                      gsems[b]).wait()
                pltpu.sync_copy(gbufs[b], agg_sh.at[dst_v.at[j]], add=True)
                nxt = j + 2

                @pl.when(nxt < _HCH)
                def _():
                    pltpu.async_copy(h_hbm.at[src_v.at[nxt]], gbufs[b],
                                     gsems[b])
            return carry

        lax.fori_loop(0, _HCH // 2, body, 0)
    plsc.subcore_barrier()
    pltpu.sync_copy(agg_sh.at[pl.ds(s * _ZROWS, _ZROWS)],
                    out_hbm.at[c, pl.ds(s * _ZROWS, _ZROWS)])


@jax.jit
def _sc_agg(h, src_p, dst_p, zeros):
    k = pl.kernel(
        _sc_agg_body,
        out_type=jax.ShapeDtypeStruct((_NSC, _AGG_ROWS, _D), jnp.float32),
        mesh=plsc.VectorSubcoreMesh(core_axis_name="c", subcore_axis_name="s"),
        scratch_types=[
            pltpu.VMEM((_HCH, _CS), jnp.int32),
            pltpu.VMEM((_HCH, _CS), jnp.int32),
            pltpu.VMEM((_CS, _D), jnp.float32),
            pltpu.VMEM((_CS, _D), jnp.float32),
            pltpu.VMEM_SHARED((_AGG_ROWS, _D), jnp.float32),
            pltpu.SemaphoreType.DMA,
            pltpu.SemaphoreType.DMA,
        ],
    )
    return k(h, src_p, dst_p, zeros)


def _bn_body(x_ref, imp_ref, g_ref, b_ref, o_ref):
    y = x_ref[...] * imp_ref[...]
    m = jnp.mean(y, axis=0, keepdims=True)
    d = y - m
    v = jnp.mean(d * d, axis=0, keepdims=True)
    o_ref[...] = d * lax.rsqrt(v + 1e-5) * g_ref[...] + b_ref[...]


def _bn(x, imp, g, b):
    return pl.pallas_call(
        _bn_body,
        out_shape=jax.ShapeDtypeStruct((_N, _D), jnp.float32),
    )(x, imp, g.reshape(1, _D), b.reshape(1, _D))


def _mlp_body(h_ref, a_ref, w_ref, b_ref, o_ref):
    x = h_ref[...] + a_ref[0, :_N] + a_ref[1, :_N]
    o_ref[...] = jnp.tanh(
        jnp.dot(x, w_ref[...], preferred_element_type=jnp.float32)
        + b_ref[...])


def _mlp(h, aggp, w, b):
    return pl.pallas_call(
        _mlp_body,
        out_shape=jax.ShapeDtypeStruct((_N, _D), jnp.float32),
    )(h, aggp, w, b.reshape(1, _D))


def _mlp_fc_body(h_ref, a_ref, w_ref, b_ref, wfc_ref, o5_ref, o6_ref):
    x = h_ref[...] + a_ref[0, :_N] + a_ref[1, :_N]
    h5 = jnp.tanh(
        jnp.dot(x, w_ref[...], preferred_element_type=jnp.float32)
        + b_ref[...])
    o5_ref[...] = h5
    o6_ref[...] = jnp.tanh(
        jnp.dot(h5, wfc_ref[...], preferred_element_type=jnp.float32))


def _mlp_fc(h, aggp, w, b, wfc):
    return pl.pallas_call(
        _mlp_fc_body,
        out_shape=(jax.ShapeDtypeStruct((_N, _D), jnp.float32),
                   jax.ShapeDtypeStruct((_N, _D), jnp.float32)),
    )(h, aggp, w, b.reshape(1, _D), wfc)


def _gin(X, imp, ei, g, b, Ws, bs, Wfc, zeros):
    pad = _EPAD - _E
    ar = jnp.arange(pad, dtype=jnp.int32)
    src_p = jnp.concatenate([ei[0], ar % _N]).reshape(_NW, _CH, _CS)
    dst_p = jnp.concatenate(
        [ei[1], _N + (ar % (_AGG_ROWS - _N))]).reshape(_NW, _CH, _CS)
    h = _bn(X, imp, g, b)
    hs = []
    for i, (W, bb) in enumerate(zip(Ws, bs)):
        aggp = _sc_agg(h, src_p, dst_p, zeros)
        if i < 4:
            h = _mlp(h, aggp, W, bb)
            hs.append(h)
        else:
            h5, h6 = _mlp_fc(h, aggp, W, bb, Wfc)
            hs.append(h5)
            hs.append(h6)
    return jnp.concatenate(hs, axis=-1)


def kernel(source_x, source_x_importance, source_edge_index, target_x,
           target_x_importance, target_edge_index, bn_gamma, bn_beta,
           W1, b1, W2, b2, W3, b3, W4, b4, W5, b5, Wfc):
    Ws = [W1, W2, W3, W4, W5]
    bs = [b1, b2, b3, b4, b5]
    zeros = jnp.zeros((_ZROWS, _D), jnp.float32)
    out_s = _gin(source_x, source_x_importance, source_edge_index,
                 bn_gamma, bn_beta, Ws, bs, Wfc, zeros)
    out_t = _gin(target_x, target_x_importance, target_edge_index,
                 bn_gamma, bn_beta, Ws, bs, Wfc, zeros)
    return (out_s, out_t)
